# baseline (speedup 1.0000x reference)
import functools

import jax
import jax.numpy as jnp
from jax import lax
from jax.experimental import pallas as pl
from jax.experimental.pallas import tpu as pltpu
from jax.experimental.pallas import tpu_sc as plsc

N = 10000
E = 320000
D = 128
HD = D // 2

CH = 128
NR = 10240
RPT = NR // 16

SNCH = 160
SBLK = 80
EPAD = 16 * SNCH * CH

DNW = 32
DNCH = 80

_mesh = plsc.VectorSubcoreMesh(core_axis_name="c", subcore_axis_name="s")


@functools.partial(
    pl.kernel,
    out_type=jax.ShapeDtypeStruct((2, 1, NR), jnp.float32),
    mesh=_mesh,
    compiler_params=pltpu.CompilerParams(use_tc_tiling_on_sc=False),
    scratch_types=[
        pltpu.VMEM((RPT,), jnp.float32),
        pltpu.VMEM((CH,), jnp.float32),
        pltpu.VMEM((DNCH, CH), jnp.int32),
        pltpu.VMEM_SHARED((NR,), jnp.float32),
        pltpu.SemaphoreType.DMA,
        pltpu.SemaphoreType.DMA,
    ],
)
def _deg_kernel(dst_hbm, out_hbm, zbuf, ones, dsti, acc, semi, sems):
    c = lax.axis_index("c")
    s = lax.axis_index("s")
    wid = s * 2 + c

    idx_cp = pltpu.async_copy(dst_hbm.at[wid], dsti, semi)

    def fill_z(i, _):
        zbuf[pl.ds(i * 16, 16)] = jnp.zeros((16,), jnp.float32)
        return 0

    lax.fori_loop(0, RPT // 16, fill_z, 0)

    def fill_o(i, _):
        ones[pl.ds(i * 16, 16)] = jnp.full((16,), 1.0, jnp.float32)
        return 0

    lax.fori_loop(0, CH // 16, fill_o, 0)

    pltpu.sync_copy(zbuf, acc.at[pl.ds(s * RPT, RPT)])
    idx_cp.wait()
    plsc.subcore_barrier()

    def body(j, _):
        pltpu.async_copy(ones, acc.at[dsti.at[j]], sems, add=True)
        return 0

    lax.fori_loop(0, DNCH, body, 0)

    def drain(j, _):
        pltpu.make_async_copy(ones, acc.at[dsti.at[j]], sems).wait()
        return 0

    lax.fori_loop(0, DNCH, drain, 0)
    plsc.subcore_barrier()
    pltpu.sync_copy(acc.at[pl.ds(s * RPT, RPT)],
                    out_hbm.at[c, 0, pl.ds(s * RPT, RPT)])


@functools.partial(
    pl.kernel,
    out_type=jax.ShapeDtypeStruct((2, NR, HD), jnp.float32),
    mesh=_mesh,
    compiler_params=pltpu.CompilerParams(use_tc_tiling_on_sc=False),
    scratch_types=[
        pltpu.VMEM((SBLK, CH), jnp.int32),
        pltpu.VMEM((SBLK, CH), jnp.int32),
        pltpu.VMEM((CH, HD), jnp.float32),
        pltpu.VMEM((CH, HD), jnp.float32),
        pltpu.VMEM_SHARED((NR, HD), jnp.float32),
        pltpu.VMEM_SHARED((NR, HD), jnp.float32),
        pltpu.SemaphoreType.DMA,
        pltpu.SemaphoreType.DMA,
        pltpu.SemaphoreType.DMA,
    ],
)
def _scatter_kernel(g_hbm, src_hbm, dst_hbm, out_hbm, srcb, dstb, r0, r1,
                    tab, acc, semi, sem0, sem1):
    c = lax.axis_index("c")
    s = lax.axis_index("s")

    pltpu.sync_copy(g_hbm.at[c, pl.ds(s * RPT, RPT)], tab.at[pl.ds(s * RPT, RPT)])
    pltpu.sync_copy(g_hbm.at[c, pl.ds(s * RPT, RPT)], acc.at[pl.ds(s * RPT, RPT)])
    plsc.subcore_barrier()

    def run_block(b):
        cp_s = pltpu.async_copy(src_hbm.at[s, pl.ds(b * SBLK, SBLK)], srcb, semi)
        cp_d = pltpu.async_copy(dst_hbm.at[s, pl.ds(b * SBLK, SBLK)], dstb, semi)
        cp_s.wait()
        cp_d.wait()
        pltpu.async_copy(tab.at[srcb.at[0]], r0, sem0)
        pltpu.async_copy(tab.at[srcb.at[1]], r1, sem1)

        def body(k, _):
            j = 2 * k
            pltpu.make_async_copy(tab.at[srcb.at[j]], r0, sem0).wait()
            pltpu.sync_copy(r0, acc.at[dstb.at[j]], add=True)
            pltpu.async_copy(tab.at[srcb.at[j + 2]], r0, sem0)
            pltpu.make_async_copy(tab.at[srcb.at[j + 1]], r1, sem1).wait()
            pltpu.sync_copy(r1, acc.at[dstb.at[j + 1]], add=True)
            pltpu.async_copy(tab.at[srcb.at[j + 3]], r1, sem1)
            return 0

        lax.fori_loop(0, SBLK // 2 - 1, body, 0)
        j = SBLK - 2
        pltpu.make_async_copy(tab.at[srcb.at[j]], r0, sem0).wait()
        pltpu.sync_copy(r0, acc.at[dstb.at[j]], add=True)
        pltpu.make_async_copy(tab.at[srcb.at[j + 1]], r1, sem1).wait()
        pltpu.sync_copy(r1, acc.at[dstb.at[j + 1]], add=True)

    for b in range(SNCH // SBLK):
        run_block(b)

    plsc.subcore_barrier()
    pltpu.sync_copy(acc.at[pl.ds(s * RPT, RPT)],
                    out_hbm.at[c, pl.ds(s * RPT, RPT)])


def _pre_body(x_ref, w_ref, gam_ref, bet_ref, deg_ref, out_ref):
    x = x_ref[...]
    mu = jnp.mean(x, axis=0, keepdims=True)
    xc = x - mu
    var = jnp.mean(xc * xc, axis=0, keepdims=True)
    h = xc * lax.rsqrt(var + 1e-5) * gam_ref[...] + bet_ref[...]
    hw = jnp.dot(h, w_ref[...], preferred_element_type=jnp.float32)
    dis = lax.rsqrt(deg_ref[...] + 1.0)
    g = hw * dis
    out_ref[0, :N] = g[:, :HD]
    out_ref[1, :N] = g[:, HD:]
    out_ref[0, N:] = jnp.zeros((NR - N, HD), jnp.float32)
    out_ref[1, N:] = jnp.zeros((NR - N, HD), jnp.float32)


def _mid_body(s_ref, deg_ref, b1_ref, m1_ref, gam_ref, bet_ref, w_ref,
              out_ref):
    dis = lax.rsqrt(deg_ref[...] + 1.0)
    sfull = jnp.concatenate([s_ref[0, :N], s_ref[1, :N]], axis=1)
    t = sfull * dis + b1_ref[...]
    t = jnp.maximum(t, 0.0) * m1_ref[...]
    mu = jnp.mean(t, axis=0, keepdims=True)
    tcen = t - mu
    var = jnp.mean(tcen * tcen, axis=0, keepdims=True)
    h = tcen * lax.rsqrt(var + 1e-5) * gam_ref[...] + bet_ref[...]
    hw = jnp.dot(h, w_ref[...], preferred_element_type=jnp.float32)
    g = hw * dis
    out_ref[0, :N] = g[:, :HD]
    out_ref[1, :N] = g[:, HD:]
    out_ref[0, N:] = jnp.zeros((NR - N, HD), jnp.float32)
    out_ref[1, N:] = jnp.zeros((NR - N, HD), jnp.float32)


def _fin_body(s_ref, deg_ref, b2_ref, m2_ref, out_ref):
    dis = lax.rsqrt(deg_ref[...] + 1.0)
    sfull = jnp.concatenate([s_ref[0, :N], s_ref[1, :N]], axis=1)
    t = sfull * dis + b2_ref[...]
    out_ref[...] = jnp.maximum(t, 0.0) * m2_ref[...]


def kernel(x, edge_index, W1, b1, gamma1, beta1, W2, b2, gamma2, beta2):
    ei = edge_index.astype(jnp.int32)
    pad = jnp.full((EPAD - E,), N, jnp.int32)
    srcp = jnp.concatenate([ei[0], pad])
    dstp = jnp.concatenate([ei[1], pad])
    src_s = srcp.reshape(16, SNCH, CH)
    dst_s = dstp.reshape(16, SNCH, CH)
    dst_d = dstp.reshape(DNW, DNCH, CH)

    dkey = jax.random.key(42)
    m1 = jnp.full((N, D), 2.0, jnp.float32) * x[0, 0]
    m2 = jnp.full((N, D), 2.0, jnp.float32) * x[0, 1]

    deg2 = _deg_kernel(dst_d)
    deg = (deg2[0, 0, :N] + deg2[1, 0, :N]).reshape(N, 1)

    g1 = pl.pallas_call(
        _pre_body,
        out_shape=jax.ShapeDtypeStruct((2, NR, HD), jnp.float32),
    )(x, W1, gamma1.reshape(1, D), beta1.reshape(1, D), deg)

    s1 = _scatter_kernel(g1, src_s, dst_s)

    g2 = pl.pallas_call(
        _mid_body,
        out_shape=jax.ShapeDtypeStruct((2, NR, HD), jnp.float32),
    )(s1, deg, b1.reshape(1, D), m1, gamma2.reshape(1, D),
      beta2.reshape(1, D), W2)

    s2 = _scatter_kernel(g2, src_s, dst_s)

    out = pl.pallas_call(
        _fin_body,
        out_shape=jax.ShapeDtypeStruct((N, D), jnp.float32),
    )(s2, deg, b2.reshape(1, D), m2)
    return out


# speedup vs baseline: 1.0140x; 1.0140x over previous
import functools

import jax
import jax.numpy as jnp
from jax import lax
from jax.experimental import pallas as pl
from jax.experimental.pallas import tpu as pltpu
from jax.experimental.pallas import tpu_sc as plsc

N = 10000
E = 320000
D = 128
HD = D // 2

CH = 128
NR = 10240
RPT = NR // 16

SNCH = 160
SBLK = 80
EPAD = 16 * SNCH * CH

DNW = 32
DNCH = 80

_mesh = plsc.VectorSubcoreMesh(core_axis_name="c", subcore_axis_name="s")


@functools.partial(
    pl.kernel,
    out_type=jax.ShapeDtypeStruct((2, 1, NR), jnp.float32),
    mesh=_mesh,
    compiler_params=pltpu.CompilerParams(use_tc_tiling_on_sc=False),
    scratch_types=[
        pltpu.VMEM((RPT,), jnp.float32),
        pltpu.VMEM((CH,), jnp.float32),
        pltpu.VMEM((DNCH, CH), jnp.int32),
        pltpu.VMEM_SHARED((NR,), jnp.float32),
        pltpu.SemaphoreType.DMA,
        pltpu.SemaphoreType.DMA,
    ],
)
def _deg_kernel(dst_hbm, out_hbm, zbuf, ones, dsti, acc, semi, sems):
    c = lax.axis_index("c")
    s = lax.axis_index("s")
    wid = s * 2 + c

    idx_cp = pltpu.async_copy(dst_hbm.at[wid], dsti, semi)

    def fill_z(i, _):
        zbuf[pl.ds(i * 16, 16)] = jnp.zeros((16,), jnp.float32)
        return 0

    lax.fori_loop(0, RPT // 16, fill_z, 0)

    def fill_o(i, _):
        ones[pl.ds(i * 16, 16)] = jnp.full((16,), 1.0, jnp.float32)
        return 0

    lax.fori_loop(0, CH // 16, fill_o, 0)

    pltpu.sync_copy(zbuf, acc.at[pl.ds(s * RPT, RPT)])
    idx_cp.wait()
    plsc.subcore_barrier()

    def body(j, _):
        pltpu.async_copy(ones, acc.at[dsti.at[j]], sems, add=True)
        return 0

    lax.fori_loop(0, DNCH, body, 0)

    def drain(j, _):
        pltpu.make_async_copy(ones, acc.at[dsti.at[j]], sems).wait()
        return 0

    lax.fori_loop(0, DNCH, drain, 0)
    plsc.subcore_barrier()
    pltpu.sync_copy(acc.at[pl.ds(s * RPT, RPT)],
                    out_hbm.at[c, 0, pl.ds(s * RPT, RPT)])


@functools.partial(
    pl.kernel,
    out_type=jax.ShapeDtypeStruct((2, NR, HD), jnp.float32),
    mesh=_mesh,
    compiler_params=pltpu.CompilerParams(use_tc_tiling_on_sc=False),
    scratch_types=[
        pltpu.VMEM((SBLK, CH), jnp.int32),
        pltpu.VMEM((SBLK, CH), jnp.int32),
        pltpu.VMEM((CH, HD), jnp.float32),
        pltpu.VMEM((CH, HD), jnp.float32),
        pltpu.VMEM_SHARED((NR, HD), jnp.float32),
        pltpu.VMEM_SHARED((NR, HD), jnp.float32),
        pltpu.SemaphoreType.DMA,
        pltpu.SemaphoreType.DMA,
        pltpu.SemaphoreType.DMA,
    ],
)
def _scatter_kernel(g_hbm, src_hbm, dst_hbm, out_hbm, srcb, dstb, r0, r1,
                    tab, acc, semi, sem0, sem1):
    c = lax.axis_index("c")
    s = lax.axis_index("s")

    pltpu.sync_copy(g_hbm.at[c, pl.ds(s * RPT, RPT)], tab.at[pl.ds(s * RPT, RPT)])
    pltpu.sync_copy(g_hbm.at[c, pl.ds(s * RPT, RPT)], acc.at[pl.ds(s * RPT, RPT)])
    plsc.subcore_barrier()

    def run_block(b):
        cp_s = pltpu.async_copy(src_hbm.at[s, pl.ds(b * SBLK, SBLK)], srcb, semi)
        cp_d = pltpu.async_copy(dst_hbm.at[s, pl.ds(b * SBLK, SBLK)], dstb, semi)
        cp_s.wait()
        cp_d.wait()
        pltpu.async_copy(tab.at[srcb.at[0]], r0, sem0)
        pltpu.async_copy(tab.at[srcb.at[1]], r1, sem1)

        def body(k, _):
            j = 2 * k
            pltpu.make_async_copy(tab.at[srcb.at[j]], r0, sem0).wait()
            pltpu.sync_copy(r0, acc.at[dstb.at[j]], add=True)
            pltpu.async_copy(tab.at[srcb.at[j + 2]], r0, sem0)
            pltpu.make_async_copy(tab.at[srcb.at[j + 1]], r1, sem1).wait()
            pltpu.sync_copy(r1, acc.at[dstb.at[j + 1]], add=True)
            pltpu.async_copy(tab.at[srcb.at[j + 3]], r1, sem1)
            return 0

        lax.fori_loop(0, SBLK // 2 - 1, body, 0)
        j = SBLK - 2
        pltpu.make_async_copy(tab.at[srcb.at[j]], r0, sem0).wait()
        pltpu.sync_copy(r0, acc.at[dstb.at[j]], add=True)
        pltpu.make_async_copy(tab.at[srcb.at[j + 1]], r1, sem1).wait()
        pltpu.sync_copy(r1, acc.at[dstb.at[j + 1]], add=True)

    for b in range(SNCH // SBLK):
        run_block(b)

    plsc.subcore_barrier()
    pltpu.sync_copy(acc.at[pl.ds(s * RPT, RPT)],
                    out_hbm.at[c, pl.ds(s * RPT, RPT)])


def _bnmm_body(x_ref, w_ref, gam_ref, bet_ref, out_ref):
    x = x_ref[...]
    mu = jnp.mean(x, axis=0, keepdims=True)
    xc = x - mu
    var = jnp.mean(xc * xc, axis=0, keepdims=True)
    h = xc * lax.rsqrt(var + 1e-5) * gam_ref[...] + bet_ref[...]
    out_ref[...] = jnp.dot(h, w_ref[...], preferred_element_type=jnp.float32)


def _scale_body(hw_ref, deg_ref, out_ref):
    dis = lax.rsqrt(deg_ref[...] + 1.0)
    g = hw_ref[...] * dis
    out_ref[0, :N] = g[:, :HD]
    out_ref[1, :N] = g[:, HD:]
    out_ref[0, N:] = jnp.zeros((NR - N, HD), jnp.float32)
    out_ref[1, N:] = jnp.zeros((NR - N, HD), jnp.float32)


def _mid_body(s_ref, deg_ref, b1_ref, m1_ref, gam_ref, bet_ref, w_ref,
              out_ref):
    dis = lax.rsqrt(deg_ref[...] + 1.0)
    sfull = jnp.concatenate([s_ref[0, :N], s_ref[1, :N]], axis=1)
    t = sfull * dis + b1_ref[...]
    t = jnp.maximum(t, 0.0) * m1_ref[...]
    mu = jnp.mean(t, axis=0, keepdims=True)
    tcen = t - mu
    var = jnp.mean(tcen * tcen, axis=0, keepdims=True)
    h = tcen * lax.rsqrt(var + 1e-5) * gam_ref[...] + bet_ref[...]
    hw = jnp.dot(h, w_ref[...], preferred_element_type=jnp.float32)
    g = hw * dis
    out_ref[0, :N] = g[:, :HD]
    out_ref[1, :N] = g[:, HD:]
    out_ref[0, N:] = jnp.zeros((NR - N, HD), jnp.float32)
    out_ref[1, N:] = jnp.zeros((NR - N, HD), jnp.float32)


def _fin_body(s_ref, deg_ref, b2_ref, m2_ref, out_ref):
    dis = lax.rsqrt(deg_ref[...] + 1.0)
    sfull = jnp.concatenate([s_ref[0, :N], s_ref[1, :N]], axis=1)
    t = sfull * dis + b2_ref[...]
    out_ref[...] = jnp.maximum(t, 0.0) * m2_ref[...]


def kernel(x, edge_index, W1, b1, gamma1, beta1, W2, b2, gamma2, beta2):
    ei = edge_index.astype(jnp.int32)
    pad = jnp.full((EPAD - E,), N, jnp.int32)
    srcp = jnp.concatenate([ei[0], pad])
    dstp = jnp.concatenate([ei[1], pad])
    src_s = srcp.reshape(16, SNCH, CH)
    dst_s = dstp.reshape(16, SNCH, CH)
    dst_d = dstp.reshape(DNW, DNCH, CH)

    dkey = jax.random.key(42)
    m1 = jax.random.bernoulli(jax.random.fold_in(dkey, 0), 0.5, (N, D))
    m2 = jax.random.bernoulli(jax.random.fold_in(dkey, 1), 0.5, (N, D))
    m1 = m1.astype(jnp.float32) * 2.0
    m2 = m2.astype(jnp.float32) * 2.0

    deg2 = _deg_kernel(dst_d)
    hw1 = pl.pallas_call(
        _bnmm_body,
        out_shape=jax.ShapeDtypeStruct((N, D), jnp.float32),
    )(x, W1, gamma1.reshape(1, D), beta1.reshape(1, D))
    deg = (deg2[0, 0, :N] + deg2[1, 0, :N]).reshape(N, 1)

    g1 = pl.pallas_call(
        _scale_body,
        out_shape=jax.ShapeDtypeStruct((2, NR, HD), jnp.float32),
    )(hw1, deg)

    s1 = _scatter_kernel(g1, src_s, dst_s)

    g2 = pl.pallas_call(
        _mid_body,
        out_shape=jax.ShapeDtypeStruct((2, NR, HD), jnp.float32),
    )(s1, deg, b1.reshape(1, D), m1, gamma2.reshape(1, D),
      beta2.reshape(1, D), W2)

    s2 = _scatter_kernel(g2, src_s, dst_s)

    out = pl.pallas_call(
        _fin_body,
        out_shape=jax.ShapeDtypeStruct((N, D), jnp.float32),
    )(s2, deg, b2.reshape(1, D), m2)
    return out
